# baseline (speedup 1.0000x reference)
import functools
import math

import jax
import jax.numpy as jnp
from jax import lax
from jax.experimental import pallas as pl
from jax.experimental.pallas import tpu as pltpu
from jax.experimental.pallas import tpu_sc as plsc

D = 64
SCALE = math.sqrt(D)
NC = 2
NS = 16
NW = NC * NS
L = 16
VOCAB = 1000000
NPAIR = VOCAB // 2
NFULL = VOCAB // 128
PITCH = 129

_mesh = lambda: plsc.VectorSubcoreMesh(core_axis_name="c", subcore_axis_name="s")
_params = lambda: pltpu.CompilerParams(
    use_tc_tiling_on_sc=True, needs_layout_passes=False)


def _transpose_kernel():
    @functools.partial(
        pl.kernel,
        mesh=_mesh(),
        compiler_params=_params(),
        out_type=jax.ShapeDtypeStruct((NPAIR, 128), jnp.float32),
        scratch_types=[
            pltpu.VMEM((D, PITCH), jnp.float32),
            pltpu.VMEM((D, PITCH), jnp.float32),
            pltpu.VMEM((D, 128), jnp.float32),
            pltpu.VMEM((D, 128), jnp.float32),
            pltpu.SemaphoreType.DMA,
            pltpu.SemaphoreType.DMA,
            pltpu.SemaphoreType.DMA,
            pltpu.SemaphoreType.DMA,
        ],
    )
    def tk(tabT, tail, tabP, in0, in1, out0, out1, gi0, gi1, go0, go1):
        ins = (in0, in1)
        outs = (out0, out1)
        gis = (gi0, gi1)
        gos = (go0, go1)
        wid = lax.axis_index("s") * NC + lax.axis_index("c")

        n_my = (NFULL - 1 - wid) // NW + 1

        def start_in(t, slot):
            b = t * NW + wid
            pltpu.async_copy(
                tabT.at[:, pl.ds(b * 128, 128)],
                ins[slot].at[:, pl.ds(0, 128)],
                gis[slot],
            )

        def wait_in(slot):
            pltpu.make_async_copy(
                tabT.at[:, pl.ds(0, 128)],
                ins[slot].at[:, pl.ds(0, 128)],
                gis[slot],
            ).wait()

        def start_out(t, slot):
            b = t * NW + wid
            pltpu.async_copy(outs[slot], tabP.at[pl.ds(b * 64, 64)], gos[slot])

        def wait_out(slot):
            pltpu.make_async_copy(
                outs[slot], tabP.at[pl.ds(0, 64)], gos[slot]).wait()

        def compute(slot):
            src = ins[slot]
            dst = outs[slot]

            def row(p, carry):
                for cb in range(8):
                    rows = lax.iota(jnp.int32, L) + (cb % 4) * L
                    cols = jnp.full((L,), 0, jnp.int32) + (2 * p + cb // 4)
                    v = plsc.load_gather(src, [rows, cols])
                    dst[p, pl.ds(cb * L, L)] = v * SCALE
                return carry

            lax.fori_loop(0, D, row, 0)

        start_in(0, 0)

        @pl.when(n_my > 1)
        def _():
            start_in(1, 1)

        def body(t, carry):
            slot = lax.rem(t, 2)

            def stage(s):
                @pl.when(t >= 2)
                def _():
                    wait_out(s)
                wait_in(s)
                compute(s)
                start_out(t, s)

                @pl.when(t + 2 < n_my)
                def _():
                    start_in(t + 2, s)

            for s in range(2):
                @pl.when(slot == s)
                def _():
                    stage(s)
            return carry

        lax.fori_loop(0, n_my, body, 0)
        wait_out(0)

        @pl.when(n_my > 1)
        def _():
            wait_out(1)

        @pl.when(wid == (NFULL % NW))
        def _():
            pltpu.sync_copy(tail, out0.at[pl.ds(0, 32)])
            pltpu.sync_copy(
                out0.at[pl.ds(0, 32)], tabP.at[pl.ds(NFULL * 64, 32)])

    return tk


def _gather_kernel():
    @functools.partial(
        pl.kernel,
        mesh=_mesh(),
        compiler_params=_params(),
        out_type=jax.ShapeDtypeStruct((200, D, 4096), jnp.float32),
        scratch_types=[
            pltpu.VMEM((8, 128), jnp.int32),
            pltpu.VMEM((8, 128), jnp.int32),
            pltpu.VMEM((128, PITCH), jnp.float32),
            pltpu.VMEM((128, PITCH), jnp.float32),
            pltpu.VMEM((D, 128), jnp.float32),
            pltpu.VMEM((D, 128), jnp.float32),
            pltpu.SemaphoreType.DMA,
            pltpu.SemaphoreType.DMA,
            pltpu.SemaphoreType.DMA,
            pltpu.SemaphoreType.DMA,
        ],
    )
    def gk(xT, tabP, out3, idxv, idx2, g0, g1, b0, b1, sg0, sg1, sb0, sb1):
        gs = (g0, g1)
        bs = (b0, b1)
        sgs = (sg0, sg1)
        sbs = (sb0, sb1)
        wid = lax.axis_index("s") * NC + lax.axis_index("c")
        col0 = wid * 128

        def load_idx_block(a):
            pltpu.sync_copy(xT.at[pl.ds(a * 8, 8), pl.ds(col0, 128)], idxv)

            def halve(r, carry):
                for q in range(8):
                    sl = pl.ds(q * L, L)
                    idx2[r, sl] = lax.shift_right_logical(idxv[r, sl], 1)
                return carry

            lax.fori_loop(0, 8, halve, 0)

        def start_gather(r, slot):
            pltpu.async_copy(
                tabP.at[idx2.at[r]], gs[slot].at[:, pl.ds(0, 128)], sgs[slot])

        def wait_gather(slot):
            pltpu.make_async_copy(
                tabP.at[idx2.at[0]], gs[slot].at[:, pl.ds(0, 128)],
                sgs[slot]).wait()

        def assemble(r, slot):
            g = gs[slot]
            b = bs[slot]

            def col(c, carry):
                for jb in range(8):
                    par = (idxv[r, pl.ds(jb * L, L)] & 1) * D
                    rows = lax.iota(jnp.int32, L) + jb * L
                    cols = par + c
                    v = plsc.load_gather(g, [rows, cols])
                    b[c, pl.ds(jb * L, L)] = v
                return carry

            lax.fori_loop(0, D, col, 0)

        def start_out(s1, slot):
            pltpu.async_copy(
                bs[slot], out3.at[s1].at[:, pl.ds(col0, 128)], sbs[slot])

        def wait_out(slot):
            pltpu.make_async_copy(
                bs[slot], out3.at[0].at[:, pl.ds(col0, 128)], sbs[slot]).wait()

        def block(a, carry):
            load_idx_block(a)
            start_gather(0, 0)
            start_gather(1, 1)

            def srow(r, carry2):
                slot = lax.rem(r, 2)

                def stage(s):
                    wait_gather(s)

                    @pl.when(r >= 2)
                    def _():
                        wait_out(s)
                    assemble(r, s)
                    start_out(a * 8 + r, s)

                    @pl.when(r + 2 < 8)
                    def _():
                        start_gather(r + 2, s)

                for s in range(2):
                    @pl.when(slot == s)
                    def _():
                        stage(s)
                return carry2

            lax.fori_loop(0, 8, srow, 0)
            wait_out(0)
            wait_out(1)
            return carry

        lax.fori_loop(0, 25, block, 0)

    return gk


def kernel(x, table):
    xT = x.T.astype(jnp.int32)
    tabT = table.T
    tail = (table[VOCAB - 64:] * SCALE).reshape(32, 128)
    tabP = _transpose_kernel()(tabT, tail)
    out3 = _gather_kernel()(xT, tabP)
    return out3.transpose(2, 0, 1)


# speedup vs baseline: 2.5160x; 2.5160x over previous
import functools
import math

import jax
import jax.numpy as jnp
from jax import lax
from jax.experimental import pallas as pl
from jax.experimental.pallas import tpu as pltpu
from jax.experimental.pallas import tpu_sc as plsc

D = 64
SCALE = math.sqrt(D)
NC = 2
NS = 16
NW = NC * NS
L = 16
VOCAB = 1000000
NPAIR = VOCAB // 2
NFULL = VOCAB // 128
PITCH = 129

_mesh = lambda: plsc.VectorSubcoreMesh(core_axis_name="c", subcore_axis_name="s")
_params = lambda: pltpu.CompilerParams(
    use_tc_tiling_on_sc=True, needs_layout_passes=False)


def _transpose_kernel():
    @functools.partial(
        pl.kernel,
        mesh=_mesh(),
        compiler_params=_params(),
        out_type=jax.ShapeDtypeStruct((NPAIR, 128), jnp.float32),
        scratch_types=[
            pltpu.VMEM((D, PITCH), jnp.float32),
            pltpu.VMEM((D, PITCH), jnp.float32),
            pltpu.VMEM((D, 128), jnp.float32),
            pltpu.VMEM((D, 128), jnp.float32),
            pltpu.SemaphoreType.DMA,
            pltpu.SemaphoreType.DMA,
            pltpu.SemaphoreType.DMA,
            pltpu.SemaphoreType.DMA,
        ],
    )
    def tk(tabT, tail, tabP, in0, in1, out0, out1, gi0, gi1, go0, go1):
        ins = (in0, in1)
        outs = (out0, out1)
        gis = (gi0, gi1)
        gos = (go0, go1)
        wid = lax.axis_index("s") * NC + lax.axis_index("c")

        n_my = (NFULL - 1 - wid) // NW + 1

        def start_in(t, slot):
            b = t * NW + wid
            pltpu.async_copy(
                tabT.at[:, pl.ds(b * 128, 128)],
                ins[slot].at[:, pl.ds(0, 128)],
                gis[slot],
            )

        def wait_in(slot):
            pltpu.make_async_copy(
                tabT.at[:, pl.ds(0, 128)],
                ins[slot].at[:, pl.ds(0, 128)],
                gis[slot],
            ).wait()

        def start_out(t, slot):
            b = t * NW + wid
            pltpu.async_copy(outs[slot], tabP.at[pl.ds(b * 64, 64)], gos[slot])

        def wait_out(slot):
            pltpu.make_async_copy(
                outs[slot], tabP.at[pl.ds(0, 64)], gos[slot]).wait()

        def compute(slot):
            src = ins[slot]
            dst = outs[slot]
            rows8 = [lax.iota(jnp.int32, L) + (cb % 4) * L for cb in range(4)]
            zero = jnp.full((L,), 0, jnp.int32)

            @plsc.parallel_loop(0, D, unroll=4)
            def _(p):
                for cb in range(8):
                    cols = zero + (2 * p + cb // 4)
                    v = plsc.load_gather(src, [rows8[cb % 4], cols])
                    dst[p, pl.ds(cb * L, L)] = v * SCALE

        start_in(0, 0)

        @pl.when(n_my > 1)
        def _():
            start_in(1, 1)

        def body(t, carry):
            slot = lax.rem(t, 2)

            def stage(s):
                @pl.when(t >= 2)
                def _():
                    wait_out(s)
                wait_in(s)
                compute(s)
                start_out(t, s)

                @pl.when(t + 2 < n_my)
                def _():
                    start_in(t + 2, s)

            for s in range(2):
                @pl.when(slot == s)
                def _():
                    stage(s)
            return carry

        lax.fori_loop(0, n_my, body, 0)
        wait_out(0)

        @pl.when(n_my > 1)
        def _():
            wait_out(1)

        @pl.when(wid == (NFULL % NW))
        def _():
            pltpu.sync_copy(tail, out0.at[pl.ds(0, 32)])
            pltpu.sync_copy(
                out0.at[pl.ds(0, 32)], tabP.at[pl.ds(NFULL * 64, 32)])

    return tk


def _gather_kernel():
    @functools.partial(
        pl.kernel,
        mesh=_mesh(),
        compiler_params=_params(),
        out_type=jax.ShapeDtypeStruct((200, D, 4096), jnp.float32),
        scratch_types=[
            pltpu.VMEM((8, 128), jnp.int32),
            pltpu.VMEM((8, 128), jnp.int32),
            pltpu.VMEM((128, PITCH), jnp.float32),
            pltpu.VMEM((128, PITCH), jnp.float32),
            pltpu.VMEM((D, 128), jnp.float32),
            pltpu.VMEM((D, 128), jnp.float32),
            pltpu.SemaphoreType.DMA,
            pltpu.SemaphoreType.DMA,
            pltpu.SemaphoreType.DMA,
            pltpu.SemaphoreType.DMA,
        ],
    )
    def gk(xT, tabP, out3, idxv, idx2, g0, g1, b0, b1, sg0, sg1, sb0, sb1):
        gs = (g0, g1)
        bs = (b0, b1)
        sgs = (sg0, sg1)
        sbs = (sb0, sb1)
        wid = lax.axis_index("s") * NC + lax.axis_index("c")
        col0 = wid * 128

        def load_idx_block(a):
            pltpu.sync_copy(xT.at[pl.ds(a * 8, 8), pl.ds(col0, 128)], idxv)

            def halve(r, carry):
                for q in range(8):
                    sl = pl.ds(q * L, L)
                    idx2[r, sl] = lax.shift_right_logical(idxv[r, sl], 1)
                return carry

            lax.fori_loop(0, 8, halve, 0)

        def start_gather(r, slot):
            pltpu.async_copy(
                tabP.at[idx2.at[r]], gs[slot].at[:, pl.ds(0, 128)], sgs[slot])

        def wait_gather(slot):
            pltpu.make_async_copy(
                tabP.at[idx2.at[0]], gs[slot].at[:, pl.ds(0, 128)],
                sgs[slot]).wait()

        def assemble(r, slot):
            g = gs[slot]
            b = bs[slot]
            for jb in range(8):
                par = (idxv[r, pl.ds(jb * L, L)] & 1) * D
                rows = lax.iota(jnp.int32, L) + jb * L

                @plsc.parallel_loop(0, D, unroll=4)
                def _(c):
                    v = plsc.load_gather(g, [rows, par + c])
                    b[c, pl.ds(jb * L, L)] = v

        def start_out(s1, slot):
            pltpu.async_copy(
                bs[slot], out3.at[s1].at[:, pl.ds(col0, 128)], sbs[slot])

        def wait_out(slot):
            pltpu.make_async_copy(
                bs[slot], out3.at[0].at[:, pl.ds(col0, 128)], sbs[slot]).wait()

        def block(a, carry):
            load_idx_block(a)
            start_gather(0, 0)
            start_gather(1, 1)

            def srow(r, carry2):
                slot = lax.rem(r, 2)

                def stage(s):
                    wait_gather(s)

                    @pl.when(r >= 2)
                    def _():
                        wait_out(s)
                    assemble(r, s)
                    start_out(a * 8 + r, s)

                    @pl.when(r + 2 < 8)
                    def _():
                        start_gather(r + 2, s)

                for s in range(2):
                    @pl.when(slot == s)
                    def _():
                        stage(s)
                return carry2

            lax.fori_loop(0, 8, srow, 0)
            wait_out(0)
            wait_out(1)
            return carry

        lax.fori_loop(0, 25, block, 0)

    return gk


def kernel(x, table):
    xT = x.T.astype(jnp.int32)
    tabT = table.T
    tail = (table[VOCAB - 64:] * SCALE).reshape(32, 128)
    tabP = _transpose_kernel()(tabT, tail)
    out3 = _gather_kernel()(xT, tabP)
    return out3.transpose(2, 0, 1)
